# baseline (speedup 1.0000x reference)
import functools
import numpy as np
import jax
import jax.numpy as jnp
from jax import lax
from jax.experimental import pallas as pl
from jax.experimental.pallas import tpu as pltpu
from jax.experimental.pallas import tpu_sc as plsc

NC = 2
NS = 16
L = 16
NW = NC * NS

B = 4096
T = 50
D = 31
DP = 32
V = 1000000
BPW = B // NW

RT_FULL = 7812
K_STEPS = 126

_HALF = np.float32(0.5)
_THREEHALF = np.float32(1.5)
_ONE = np.float32(1.0)


def _rsqrt(x):
    i = plsc.bitcast(x, jnp.int32)
    i = jnp.int32(0x5F3759DF) - (i >> 1)
    y = plsc.bitcast(i, jnp.float32)
    for _ in range(3):
        y = y * (_THREEHALF - _HALF * x * y * y)
    return y


def _retile_body(tab_hbm, tail_hbm, lin_hbm, vin, vout, tailv, sem_i0,
                 sem_i1, sem_i2, sem_u0, sem_u1, sem_u2):
    wid = lax.axis_index("s") * NC + lax.axis_index("c")
    sem_i = (sem_i0, sem_i1, sem_i2)
    sem_u = (sem_u0, sem_u1, sem_u2)
    iota = lax.iota(jnp.int32, L)

    start_w = wid * 244 + jnp.minimum(wid, 4)

    def rt_of(k):
        return jnp.minimum(start_w + 2 * k, RT_FULL - 2)

    c_lo = iota
    c_hi = iota + L

    def fire_in(k, p):
        rt = rt_of(k)
        r0 = pl.multiple_of(rt * 128, 128)
        for ct in range(4):
            c = 8 if ct < 3 else 7
            pltpu.async_copy(tab_hbm.at[pl.ds(8 * ct, c), pl.ds(r0, 256)],
                             vin.at[p, pl.ds(8 * ct, c), pl.ds(0, 256)],
                             sem_i[p])

    def drain_in(p):
        for ct in range(4):
            c = 8 if ct < 3 else 7
            pltpu.make_async_copy(tab_hbm.at[pl.ds(0, c), pl.ds(0, 256)],
                                  vin.at[p, pl.ds(8 * ct, c),
                                         pl.ds(0, 256)],
                                  sem_i[p]).wait()

    def transpose(p, nrows):
        def rowblk(q, carry):
            base = q * 8
            for u in range(8):
                rr = base + u
                rsplat = jnp.full((L,), 0, jnp.int32) + rr
                lo = plsc.load_gather(vin.at[p], [c_lo, rsplat])
                hi = plsc.load_gather(vin.at[p], [c_hi, rsplat])
                vout[p, rr >> 5, (rr >> 2) & 7,
                     pl.ds((rr & 3) * 32, L)] = lo
                vout[p, rr >> 5, (rr >> 2) & 7,
                     pl.ds((rr & 3) * 32 + L, L)] = hi
            return carry

        lax.fori_loop(0, nrows // 8, rowblk, 0, unroll=False)

    def fire_out(k, p):
        rt = rt_of(k)
        pltpu.async_copy(vout.at[p], lin_hbm.at[pl.ds(4 * rt, 8)],
                         sem_u[p])

    def drain_out(p):
        pltpu.make_async_copy(vout.at[p], lin_hbm.at[pl.ds(0, 8)],
                              sem_u[p]).wait()

    fire_in(0, 0)
    fire_in(1, 1)

    def step(j, carry):
        for off in range(3):
            p = off
            k = 3 * j + off
            kn = jnp.minimum(k + 2, K_STEPS - 1)
            fire_in(kn, (off + 2) % 3)
            drain_in(p)

            @pl.when(k >= 3)
            def _():
                drain_out(p)

            transpose(p, 256)
            fire_out(k, p)
        return carry

    lax.fori_loop(0, K_STEPS // 3, step, 0, unroll=False)
    drain_in(0)
    drain_in(1)
    drain_out(0)
    drain_out(1)
    drain_out(2)

    @pl.when(wid == NW - 1)
    def _():
        pltpu.sync_copy(tail_hbm, tailv)
        for rr in range(64):
            rsplat = jnp.full((L,), rr, jnp.int32)
            lo = plsc.load_gather(tailv, [c_lo, rsplat])
            hi = plsc.load_gather(tailv, [c_hi, rsplat])
            vout[0, rr >> 5, (rr >> 2) & 7, pl.ds((rr & 3) * 32, L)] = lo
            vout[0, rr >> 5, (rr >> 2) & 7,
                 pl.ds((rr & 3) * 32 + L, L)] = hi
        pltpu.sync_copy(vout.at[0, pl.ds(0, 2)],
                        lin_hbm.at[pl.ds(4 * RT_FULL, 2)])


def _gather_body(idx_hbm, lin_hbm, out_hbm, idxv, dstv, dsts, outv,
                 sem_g0, sem_g1, sem_o0, sem_o1):
    wid = lax.axis_index("s") * NC + lax.axis_index("c")
    col0 = pl.multiple_of(wid * BPW, BPW)
    sem_g = (sem_g0, sem_g1)
    sem_o = (sem_o0, sem_o1)
    lanes_iota = lax.iota(jnp.int32, L)

    pltpu.sync_copy(idx_hbm.at[:, pl.ds(col0, BPW)], idxv)

    def fire_gather(t, p):
        pltpu.async_copy(lin_hbm.at[idxv.at[t]], dstv.at[p], sem_g[p])

    def drain_gather(p):
        pltpu.make_async_copy(lin_hbm.at[idxv.at[0]], dstv.at[p],
                              sem_g[p]).wait()

    def repack(p):
        def blk(q, carry):
            for u in range(8):
                rr = q * 8 + u
                lo = dstv[p, rr, pl.ds(0, L)]
                hi = dstv[p, rr, pl.ds(L, L)]
                dsts[p, rr, pl.ds(0, L)] = lo
                dsts[p, rr, pl.ds(L, L)] = hi
            return carry

        lax.fori_loop(0, BPW // 8, blk, 0, unroll=False)

    def compute(t, p):
        repack(p)
        for bb in range(8):
            lanes = pl.ds(bb * L, L)
            rows = lanes_iota + bb * L
            vs = [plsc.load_gather(dsts.at[p], [rows, jnp.full(
                (L,), c, jnp.int32)]) for c in range(D)]
            s = vs[0] * vs[0]
            for d in range(1, D):
                s = s + vs[d] * vs[d]
            s_c = jnp.maximum(s, jnp.float32(1e-30))
            norm = jnp.maximum(s_c * _rsqrt(s_c), jnp.float32(1e-8))
            e = jnp.exp(norm)
            scale_big = (e - _ONE / e) / (norm + norm)
            n2 = norm * norm
            scale_small = _ONE + n2 * (jnp.float32(1.0 / 6.0)
                                       + n2 * jnp.float32(1.0 / 120.0))
            scale = jnp.where(norm < jnp.float32(0.1), scale_small,
                              scale_big)
            tt = scale * scale * s + _ONE
            x0 = jnp.maximum(tt * _rsqrt(tt), _ONE)
            outv[p, 0, 0, lanes] = x0
            for d in range(D):
                q = d + 1
                outv[p, q // 8, q % 8, lanes] = scale * vs[d]

    def fire_out(t, p):
        for dt in range(4):
            pltpu.async_copy(outv.at[p, dt], out_hbm.at[t, dt, wid],
                             sem_o[p])

    def drain_out(p):
        for dt in range(4):
            pltpu.make_async_copy(outv.at[p, dt], out_hbm.at[0, dt, 0],
                                  sem_o[p]).wait()

    fire_gather(0, 0)

    def step(i, carry):
        for off in range(2):
            p = off
            t = 2 * i + off
            tn = jnp.minimum(t + 1, T - 1)
            fire_gather(tn, 1 - p)
            drain_gather(p)

            @pl.when(t >= 2)
            def _():
                drain_out(p)

            compute(t, p)
            fire_out(t, p)
        return carry

    lax.fori_loop(0, T // 2, step, 0, unroll=False)
    drain_gather(0)
    drain_out(0)
    drain_out(1)


@jax.jit
def _run(idx, tangent):
    idxT = idx.T.astype(jnp.int32)
    tabT = tangent.T
    tailp = jnp.pad(lax.slice(tabT, (0, RT_FULL * 128), (D, V)),
                    ((0, 1), (0, 64)))
    mesh = plsc.VectorSubcoreMesh(core_axis_name="c", subcore_axis_name="s")
    retile = pl.kernel(
        _retile_body,
        out_type=jax.ShapeDtypeStruct((V * DP // 1024, 8, 128),
                                      jnp.float32),
        mesh=mesh,
        scratch_types=[
            pltpu.VMEM((3, 32, 257), jnp.float32),
            pltpu.VMEM((3, 8, 8, 128), jnp.float32),
            pltpu.VMEM((32, 128), jnp.float32),
            pltpu.SemaphoreType.DMA,
            pltpu.SemaphoreType.DMA,
            pltpu.SemaphoreType.DMA,
            pltpu.SemaphoreType.DMA,
            pltpu.SemaphoreType.DMA,
            pltpu.SemaphoreType.DMA,
        ],
        compiler_params=pltpu.CompilerParams(
            needs_layout_passes=False,
            use_tc_tiling_on_sc=True,
            disable_bounds_checks=True,
        ),
    )
    lin = retile(tabT, tailp)
    lin2 = jnp.reshape(lin, (V, DP))

    gather = pl.kernel(
        _gather_body,
        out_type=jax.ShapeDtypeStruct((T, 4, NW, 8, BPW), jnp.float32),
        mesh=mesh,
        scratch_types=[
            pltpu.VMEM((T, BPW), jnp.int32),
            pltpu.VMEM((2, BPW, DP), jnp.float32),
            pltpu.VMEM((2, BPW, DP + 1), jnp.float32),
            pltpu.VMEM((2, 4, 8, BPW), jnp.float32),
            pltpu.SemaphoreType.DMA,
            pltpu.SemaphoreType.DMA,
            pltpu.SemaphoreType.DMA,
            pltpu.SemaphoreType.DMA,
        ],
        compiler_params=pltpu.CompilerParams(
            needs_layout_passes=False,
            use_tc_tiling_on_sc=False,
            disable_bounds_checks=True,
        ),
    )
    out = gather(idxT, lin2)
    return out.transpose(2, 4, 0, 1, 3).reshape(B, T, D + 1)


def kernel(idx, tangent):
    return _run(idx, tangent)


# speedup vs baseline: 1.3426x; 1.3426x over previous
import functools
import numpy as np
import jax
import jax.numpy as jnp
from jax import lax
from jax.experimental import pallas as pl
from jax.experimental.pallas import tpu as pltpu
from jax.experimental.pallas import tpu_sc as plsc

NC = 2
NS = 16
L = 16
NW = NC * NS

B = 4096
T = 50
D = 31
DP = 32
V = 1000000
BPW = B // NW

RT_FULL = 7812
K_STEPS = 126

_HALF = np.float32(0.5)
_THREEHALF = np.float32(1.5)
_ONE = np.float32(1.0)


def _rsqrt(x):
    i = plsc.bitcast(x, jnp.int32)
    i = jnp.int32(0x5F3759DF) - (i >> 1)
    y = plsc.bitcast(i, jnp.float32)
    for _ in range(3):
        y = y * (_THREEHALF - _HALF * x * y * y)
    return y


def _retile_body(tab_hbm, tail_hbm, lin_hbm, vin, vout, tailv, sem_i0,
                 sem_i1, sem_i2, sem_u0, sem_u1, sem_u2):
    wid = lax.axis_index("s") * NC + lax.axis_index("c")
    sem_i = (sem_i0, sem_i1, sem_i2)
    sem_u = (sem_u0, sem_u1, sem_u2)
    iota = lax.iota(jnp.int32, L)

    start_w = wid * 244 + jnp.minimum(wid, 4)

    def rt_of(k):
        return jnp.minimum(start_w + 2 * k, RT_FULL - 2)

    def fire_in(k, p):
        rt = rt_of(k)
        r0 = pl.multiple_of(rt * 128, 128)
        for ct in range(4):
            c = 8 if ct < 3 else 7
            pltpu.async_copy(tab_hbm.at[pl.ds(8 * ct, c), pl.ds(r0, 256)],
                             vin.at[p, ct, pl.ds(0, c)], sem_i[p])

    def drain_in(p):
        for ct in range(4):
            c = 8 if ct < 3 else 7
            pltpu.make_async_copy(tab_hbm.at[pl.ds(0, c), pl.ds(0, 256)],
                                  vin.at[p, ct, pl.ds(0, c)],
                                  sem_i[p]).wait()

    def transpose(p, ngroups):
        for g in range(ngroups):
            rr = iota + g * L
            a = (g * L) >> 5
            bvec = (rr >> 2) & 7
            dbase = (rr & 3) << 5
            for ct in range(4):
                cmax = 8 if ct < 3 else 7
                vals = [vin[p, ct, cl, pl.ds(g * L, L)]
                        for cl in range(cmax)]
                for cl in range(cmax):
                    plsc.store_scatter(vout.at[p, a],
                                       [bvec, dbase + (8 * ct + cl)],
                                       vals[cl])

    def fire_out(k, p):
        rt = rt_of(k)
        pltpu.async_copy(vout.at[p], lin_hbm.at[pl.ds(4 * rt, 8)],
                         sem_u[p])

    def drain_out(p):
        pltpu.make_async_copy(vout.at[p], lin_hbm.at[pl.ds(0, 8)],
                              sem_u[p]).wait()

    fire_in(0, 0)
    fire_in(1, 1)

    def step(j, carry):
        for off in range(3):
            p = off
            k = 3 * j + off
            kn = jnp.minimum(k + 2, K_STEPS - 1)
            fire_in(kn, (off + 2) % 3)
            drain_in(p)

            @pl.when(k >= 3)
            def _():
                drain_out(p)

            transpose(p, 16)
            fire_out(k, p)
        return carry

    lax.fori_loop(0, K_STEPS // 3, step, 0, unroll=False)
    drain_in(0)
    drain_in(1)
    drain_out(0)
    drain_out(1)
    drain_out(2)

    @pl.when(wid == NW - 1)
    def _():
        pltpu.sync_copy(tail_hbm, tailv)
        for g in range(4):
            rr = iota + g * L
            a = (g * L) >> 5
            bvec = (rr >> 2) & 7
            dbase = (rr & 3) << 5
            for c in range(D):
                v = tailv[c, pl.ds(g * L, L)]
                plsc.store_scatter(vout.at[0, a], [bvec, dbase + c], v)
        pltpu.sync_copy(vout.at[0, pl.ds(0, 2)],
                        lin_hbm.at[pl.ds(4 * RT_FULL, 2)])


def _gather_body(idx_hbm, lin_hbm, out_hbm, idxv, dstv, dsts, outv,
                 sem_g0, sem_g1, sem_o0, sem_o1):
    wid = lax.axis_index("s") * NC + lax.axis_index("c")
    col0 = pl.multiple_of(wid * BPW, BPW)
    sem_g = (sem_g0, sem_g1)
    sem_o = (sem_o0, sem_o1)
    lanes_iota = lax.iota(jnp.int32, L)

    pltpu.sync_copy(idx_hbm.at[:, pl.ds(col0, BPW)], idxv)

    def fire_gather(t, p):
        pltpu.async_copy(lin_hbm.at[idxv.at[t]], dstv.at[p], sem_g[p])

    def drain_gather(p):
        pltpu.make_async_copy(lin_hbm.at[idxv.at[0]], dstv.at[p],
                              sem_g[p]).wait()

    def repack(p):
        def blk(q, carry):
            for u in range(8):
                rr = q * 8 + u
                lo = dstv[p, rr, pl.ds(0, L)]
                hi = dstv[p, rr, pl.ds(L, L)]
                dsts[p, rr, pl.ds(0, L)] = lo
                dsts[p, rr, pl.ds(L, L)] = hi
            return carry

        lax.fori_loop(0, BPW // 8, blk, 0, unroll=False)

    def compute(t, p):
        repack(p)
        for bb in range(8):
            lanes = pl.ds(bb * L, L)
            rows = lanes_iota + bb * L
            vs = [plsc.load_gather(dsts.at[p], [rows, jnp.full(
                (L,), c, jnp.int32)]) for c in range(D)]
            s = vs[0] * vs[0]
            for d in range(1, D):
                s = s + vs[d] * vs[d]
            s_c = jnp.maximum(s, jnp.float32(1e-30))
            norm = jnp.maximum(s_c * _rsqrt(s_c), jnp.float32(1e-8))
            e = jnp.exp(norm)
            scale_big = (e - _ONE / e) / (norm + norm)
            n2 = norm * norm
            scale_small = _ONE + n2 * (jnp.float32(1.0 / 6.0)
                                       + n2 * jnp.float32(1.0 / 120.0))
            scale = jnp.where(norm < jnp.float32(0.1), scale_small,
                              scale_big)
            tt = scale * scale * s + _ONE
            x0 = jnp.maximum(tt * _rsqrt(tt), _ONE)
            outv[p, 0, 0, lanes] = x0
            for d in range(D):
                q = d + 1
                outv[p, q // 8, q % 8, lanes] = scale * vs[d]

    def fire_out(t, p):
        for dt in range(4):
            pltpu.async_copy(outv.at[p, dt], out_hbm.at[t, dt, wid],
                             sem_o[p])

    def drain_out(p):
        for dt in range(4):
            pltpu.make_async_copy(outv.at[p, dt], out_hbm.at[0, dt, 0],
                                  sem_o[p]).wait()

    fire_gather(0, 0)

    def step(i, carry):
        for off in range(2):
            p = off
            t = 2 * i + off
            tn = jnp.minimum(t + 1, T - 1)
            fire_gather(tn, 1 - p)
            drain_gather(p)

            @pl.when(t >= 2)
            def _():
                drain_out(p)

            compute(t, p)
            fire_out(t, p)
        return carry

    lax.fori_loop(0, T // 2, step, 0, unroll=False)
    drain_gather(0)
    drain_out(0)
    drain_out(1)


@jax.jit
def _run(idx, tangent):
    idxT = idx.T.astype(jnp.int32)
    tabT = tangent.T
    tailp = jnp.pad(lax.slice(tabT, (0, RT_FULL * 128), (D, V)),
                    ((0, 1), (0, 64)))
    mesh = plsc.VectorSubcoreMesh(core_axis_name="c", subcore_axis_name="s")
    retile = pl.kernel(
        _retile_body,
        out_type=jax.ShapeDtypeStruct((V * DP // 1024, 8, 128),
                                      jnp.float32),
        mesh=mesh,
        scratch_types=[
            pltpu.VMEM((3, 4, 8, 256), jnp.float32),
            pltpu.VMEM((3, 8, 8, 128), jnp.float32),
            pltpu.VMEM((32, 128), jnp.float32),
            pltpu.SemaphoreType.DMA,
            pltpu.SemaphoreType.DMA,
            pltpu.SemaphoreType.DMA,
            pltpu.SemaphoreType.DMA,
            pltpu.SemaphoreType.DMA,
            pltpu.SemaphoreType.DMA,
        ],
        compiler_params=pltpu.CompilerParams(
            needs_layout_passes=False,
            use_tc_tiling_on_sc=True,
            disable_bounds_checks=True,
        ),
    )
    lin = retile(tabT, tailp)
    lin2 = jnp.reshape(lin, (V, DP))

    gather = pl.kernel(
        _gather_body,
        out_type=jax.ShapeDtypeStruct((T, 4, NW, 8, BPW), jnp.float32),
        mesh=mesh,
        scratch_types=[
            pltpu.VMEM((T, BPW), jnp.int32),
            pltpu.VMEM((2, BPW, DP), jnp.float32),
            pltpu.VMEM((2, BPW, DP + 1), jnp.float32),
            pltpu.VMEM((2, 4, 8, BPW), jnp.float32),
            pltpu.SemaphoreType.DMA,
            pltpu.SemaphoreType.DMA,
            pltpu.SemaphoreType.DMA,
            pltpu.SemaphoreType.DMA,
        ],
        compiler_params=pltpu.CompilerParams(
            needs_layout_passes=False,
            use_tc_tiling_on_sc=False,
            disable_bounds_checks=True,
        ),
    )
    out = gather(idxT, lin2)
    return out.transpose(2, 4, 0, 1, 3).reshape(B, T, D + 1)


def kernel(idx, tangent):
    return _run(idx, tangent)


# speedup vs baseline: 1.3565x; 1.0104x over previous
import functools
import numpy as np
import jax
import jax.numpy as jnp
from jax import lax
from jax.experimental import pallas as pl
from jax.experimental.pallas import tpu as pltpu
from jax.experimental.pallas import tpu_sc as plsc

NC = 2
NS = 16
L = 16
NW = NC * NS

B = 4096
T = 50
D = 31
DP = 32
V = 1000000
BPW = B // NW

RT_FULL = 7812
K_STEPS = 62

_HALF = np.float32(0.5)
_THREEHALF = np.float32(1.5)
_ONE = np.float32(1.0)


def _rsqrt(x):
    i = plsc.bitcast(x, jnp.int32)
    i = jnp.int32(0x5F3759DF) - (i >> 1)
    y = plsc.bitcast(i, jnp.float32)
    for _ in range(3):
        y = y * (_THREEHALF - _HALF * x * y * y)
    return y


def _retile_body(tab_hbm, tail_hbm, lin_hbm, vin, vout, tailv, sem_i0,
                 sem_i1, sem_i2, sem_u0, sem_u1, sem_u2):
    wid = lax.axis_index("s") * NC + lax.axis_index("c")
    sem_i = (sem_i0, sem_i1, sem_i2)
    sem_u = (sem_u0, sem_u1, sem_u2)
    iota = lax.iota(jnp.int32, L)

    start_w = wid * 244 + jnp.minimum(wid, 4)

    def rt_of(k):
        return jnp.minimum(start_w + 4 * k, RT_FULL - 4)

    def fire_in(k, p):
        rt = rt_of(k)
        r0 = pl.multiple_of(rt * 128, 128)
        for ct in range(4):
            c = 8 if ct < 3 else 7
            pltpu.async_copy(tab_hbm.at[pl.ds(8 * ct, c), pl.ds(r0, 512)],
                             vin.at[p, ct, pl.ds(0, c)], sem_i[p])

    def drain_in(p):
        for ct in range(4):
            c = 8 if ct < 3 else 7
            pltpu.make_async_copy(tab_hbm.at[pl.ds(0, c), pl.ds(0, 512)],
                                  vin.at[p, ct, pl.ds(0, c)],
                                  sem_i[p]).wait()

    def transpose(p, ngroups):
        for g in range(ngroups):
            rr = iota + g * L
            a = (g * L) >> 5
            bvec = (rr >> 2) & 7
            dbase = (rr & 3) << 5
            for ct in range(4):
                cmax = 8 if ct < 3 else 7
                vals = [vin[p, ct, cl, pl.ds(g * L, L)]
                        for cl in range(cmax)]
                for cl in range(cmax):
                    plsc.store_scatter(vout.at[p, a],
                                       [bvec, dbase + (8 * ct + cl)],
                                       vals[cl])

    def fire_out(k, p):
        rt = rt_of(k)
        pltpu.async_copy(vout.at[p], lin_hbm.at[pl.ds(4 * rt, 16)],
                         sem_u[p])

    def drain_out(p):
        pltpu.make_async_copy(vout.at[p], lin_hbm.at[pl.ds(0, 16)],
                              sem_u[p]).wait()

    fire_in(0, 0)

    def step(j, carry):
        for off in range(2):
            p = off
            k = 2 * j + off
            kn = jnp.minimum(k + 1, K_STEPS - 1)
            fire_in(kn, 1 - p)
            drain_in(p)

            @pl.when(k >= 2)
            def _():
                drain_out(p)

            transpose(p, 32)
            fire_out(k, p)
        return carry

    lax.fori_loop(0, K_STEPS // 2, step, 0, unroll=False)
    drain_in(0)
    drain_out(0)
    drain_out(1)

    @pl.when(wid == NW - 1)
    def _():
        pltpu.sync_copy(tail_hbm, tailv)
        for g in range(4):
            rr = iota + g * L
            a = (g * L) >> 5
            bvec = (rr >> 2) & 7
            dbase = (rr & 3) << 5
            for c in range(D):
                v = tailv[c, pl.ds(g * L, L)]
                plsc.store_scatter(vout.at[0, a], [bvec, dbase + c], v)
        pltpu.sync_copy(vout.at[0, pl.ds(0, 2)],
                        lin_hbm.at[pl.ds(4 * RT_FULL, 2)])


def _gather_body(idx_hbm, lin_hbm, out_hbm, idxv, dstv, dsts, outv,
                 sem_g0, sem_g1, sem_o0, sem_o1):
    wid = lax.axis_index("s") * NC + lax.axis_index("c")
    col0 = pl.multiple_of(wid * BPW, BPW)
    sem_g = (sem_g0, sem_g1)
    sem_o = (sem_o0, sem_o1)
    lanes_iota = lax.iota(jnp.int32, L)

    pltpu.sync_copy(idx_hbm.at[:, pl.ds(col0, BPW)], idxv)

    def fire_gather(t, p):
        pltpu.async_copy(lin_hbm.at[idxv.at[t]], dstv.at[p], sem_g[p])

    def drain_gather(p):
        pltpu.make_async_copy(lin_hbm.at[idxv.at[0]], dstv.at[p],
                              sem_g[p]).wait()

    def repack(p):
        def blk(q, carry):
            for u in range(8):
                rr = q * 8 + u
                lo = dstv[p, rr, pl.ds(0, L)]
                hi = dstv[p, rr, pl.ds(L, L)]
                dsts[p, rr, pl.ds(0, L)] = lo
                dsts[p, rr, pl.ds(L, L)] = hi
            return carry

        lax.fori_loop(0, BPW // 8, blk, 0, unroll=False)

    def compute(t, p):
        repack(p)
        for bb in range(8):
            lanes = pl.ds(bb * L, L)
            rows = lanes_iota + bb * L
            vs = [plsc.load_gather(dsts.at[p], [rows, jnp.full(
                (L,), c, jnp.int32)]) for c in range(D)]
            s = vs[0] * vs[0]
            for d in range(1, D):
                s = s + vs[d] * vs[d]
            s_c = jnp.maximum(s, jnp.float32(1e-30))
            norm = jnp.maximum(s_c * _rsqrt(s_c), jnp.float32(1e-8))
            e = jnp.exp(norm)
            scale_big = (e - _ONE / e) / (norm + norm)
            n2 = norm * norm
            scale_small = _ONE + n2 * (jnp.float32(1.0 / 6.0)
                                       + n2 * jnp.float32(1.0 / 120.0))
            scale = jnp.where(norm < jnp.float32(0.1), scale_small,
                              scale_big)
            tt = scale * scale * s + _ONE
            x0 = jnp.maximum(tt * _rsqrt(tt), _ONE)
            outv[p, 0, 0, lanes] = x0
            for d in range(D):
                q = d + 1
                outv[p, q // 8, q % 8, lanes] = scale * vs[d]

    def fire_out(t, p):
        for dt in range(4):
            pltpu.async_copy(outv.at[p, dt], out_hbm.at[t, dt, wid],
                             sem_o[p])

    def drain_out(p):
        for dt in range(4):
            pltpu.make_async_copy(outv.at[p, dt], out_hbm.at[0, dt, 0],
                                  sem_o[p]).wait()

    fire_gather(0, 0)

    def step(i, carry):
        for off in range(2):
            p = off
            t = 2 * i + off
            tn = jnp.minimum(t + 1, T - 1)
            fire_gather(tn, 1 - p)
            drain_gather(p)

            @pl.when(t >= 2)
            def _():
                drain_out(p)

            compute(t, p)
            fire_out(t, p)
        return carry

    lax.fori_loop(0, T // 2, step, 0, unroll=False)
    drain_gather(0)
    drain_out(0)
    drain_out(1)


@jax.jit
def _run(idx, tangent):
    idxT = idx.T.astype(jnp.int32)
    tabT = tangent.T
    tailp = jnp.pad(lax.slice(tabT, (0, RT_FULL * 128), (D, V)),
                    ((0, 1), (0, 64)))
    mesh = plsc.VectorSubcoreMesh(core_axis_name="c", subcore_axis_name="s")
    retile = pl.kernel(
        _retile_body,
        out_type=jax.ShapeDtypeStruct((V * DP // 1024, 8, 128),
                                      jnp.float32),
        mesh=mesh,
        scratch_types=[
            pltpu.VMEM((2, 4, 8, 512), jnp.float32),
            pltpu.VMEM((2, 16, 8, 128), jnp.float32),
            pltpu.VMEM((32, 128), jnp.float32),
            pltpu.SemaphoreType.DMA,
            pltpu.SemaphoreType.DMA,
            pltpu.SemaphoreType.DMA,
            pltpu.SemaphoreType.DMA,
            pltpu.SemaphoreType.DMA,
            pltpu.SemaphoreType.DMA,
        ],
        compiler_params=pltpu.CompilerParams(
            needs_layout_passes=False,
            use_tc_tiling_on_sc=True,
            disable_bounds_checks=True,
        ),
    )
    lin = retile(tabT, tailp)
    lin2 = jnp.reshape(lin, (V, DP))

    gather = pl.kernel(
        _gather_body,
        out_type=jax.ShapeDtypeStruct((T, 4, NW, 8, BPW), jnp.float32),
        mesh=mesh,
        scratch_types=[
            pltpu.VMEM((T, BPW), jnp.int32),
            pltpu.VMEM((2, BPW, DP), jnp.float32),
            pltpu.VMEM((2, BPW, DP + 1), jnp.float32),
            pltpu.VMEM((2, 4, 8, BPW), jnp.float32),
            pltpu.SemaphoreType.DMA,
            pltpu.SemaphoreType.DMA,
            pltpu.SemaphoreType.DMA,
            pltpu.SemaphoreType.DMA,
        ],
        compiler_params=pltpu.CompilerParams(
            needs_layout_passes=False,
            use_tc_tiling_on_sc=False,
            disable_bounds_checks=True,
        ),
    )
    out = gather(idxT, lin2)
    return out.transpose(2, 4, 0, 1, 3).reshape(B, T, D + 1)


def kernel(idx, tangent):
    return _run(idx, tangent)
